# initial kernel scaffold (unmeasured)
import jax
import jax.numpy as jnp
from jax import lax
from jax.experimental import pallas as pl
from jax.experimental.pallas import tpu as pltpu


def kernel(
    x,
):
    def body(*refs):
        pass

    out_shape = jax.ShapeDtypeStruct(..., jnp.float32)
    return pl.pallas_call(body, out_shape=out_shape)(...)


# baseline (device time: 19884 ns/iter reference)
import jax
import jax.numpy as jnp
from jax import lax
from jax.experimental import pallas as pl
from jax.experimental.pallas import tpu as pltpu


def kernel(x):
    m, n = x.shape

    def body(x_ref, out_ref, send_ref, comm_ref, send_sems, recv_sems):
        my_x = lax.axis_index("x")
        my_y = lax.axis_index("y")
        x_nbr = (1 - my_x, my_y)
        y_nbr = (my_x, 1 - my_y)

        barrier_sem = pltpu.get_barrier_semaphore()
        for nbr in (x_nbr, y_nbr):
            pl.semaphore_signal(
                barrier_sem, inc=1,
                device_id=nbr, device_id_type=pl.DeviceIdType.MESH,
            )
        pl.semaphore_wait(barrier_sem, 2)

        send_ref[...] = x_ref[...].astype(jnp.bfloat16)
        rdma1 = pltpu.make_async_remote_copy(
            src_ref=send_ref,
            dst_ref=comm_ref.at[0],
            send_sem=send_sems.at[0],
            recv_sem=recv_sems.at[0],
            device_id=x_nbr,
            device_id_type=pl.DeviceIdType.MESH,
        )
        rdma1.start()
        rdma1.wait()
        out_ref[...] = x_ref[...] + comm_ref[0].astype(jnp.float32)

        send_ref[...] = out_ref[...].astype(jnp.bfloat16)
        rdma2 = pltpu.make_async_remote_copy(
            src_ref=send_ref,
            dst_ref=comm_ref.at[1],
            send_sem=send_sems.at[1],
            recv_sem=recv_sems.at[1],
            device_id=y_nbr,
            device_id_type=pl.DeviceIdType.MESH,
        )
        rdma2.start()
        rdma2.wait()
        out_ref[...] = out_ref[...] + comm_ref[1].astype(jnp.float32)

    return pl.pallas_call(
        body,
        out_shape=jax.ShapeDtypeStruct((m, n), jnp.float32),
        in_specs=[pl.BlockSpec(memory_space=pltpu.VMEM)],
        out_specs=pl.BlockSpec(memory_space=pltpu.VMEM),
        scratch_shapes=[
            pltpu.VMEM((m, n), jnp.bfloat16),
            pltpu.VMEM((2, m, n), jnp.bfloat16),
            pltpu.SemaphoreType.DMA((2,)),
            pltpu.SemaphoreType.DMA((2,)),
        ],
        compiler_params=pltpu.CompilerParams(collective_id=0),
    )(x)


# device time: 14288 ns/iter; 1.3917x vs baseline; 1.3917x over previous
import jax
import jax.numpy as jnp
from jax import lax
from jax.experimental import pallas as pl
from jax.experimental.pallas import tpu as pltpu


def kernel(x):
    m, n = x.shape
    h = m // 2

    def body(x_ref, out_ref, send_ref, comm_ref, send_sems, recv_sems):
        my_x = lax.axis_index("x")
        my_y = lax.axis_index("y")
        x_nbr = (1 - my_x, my_y)
        y_nbr = (my_x, 1 - my_y)

        barrier_sem = pltpu.get_barrier_semaphore()
        for nbr in (x_nbr, y_nbr):
            pl.semaphore_signal(
                barrier_sem, inc=1,
                device_id=nbr, device_id_type=pl.DeviceIdType.MESH,
            )
        pl.semaphore_wait(barrier_sem, 2)

        send_ref[0] = x_ref[pl.ds(0, h), :].astype(jnp.bfloat16)
        send_ref[1] = x_ref[pl.ds(h, h), :].astype(jnp.bfloat16)
        rdma_a1 = pltpu.make_async_remote_copy(
            src_ref=send_ref.at[0],
            dst_ref=comm_ref.at[0],
            send_sem=send_sems.at[0],
            recv_sem=recv_sems.at[0],
            device_id=x_nbr,
            device_id_type=pl.DeviceIdType.MESH,
        )
        rdma_b1 = pltpu.make_async_remote_copy(
            src_ref=send_ref.at[1],
            dst_ref=comm_ref.at[1],
            send_sem=send_sems.at[1],
            recv_sem=recv_sems.at[1],
            device_id=y_nbr,
            device_id_type=pl.DeviceIdType.MESH,
        )
        rdma_a1.start()
        rdma_b1.start()
        rdma_a1.wait()
        rdma_b1.wait()
        out_ref[pl.ds(0, h), :] = (
            x_ref[pl.ds(0, h), :] + comm_ref[0].astype(jnp.float32)
        )
        out_ref[pl.ds(h, h), :] = (
            x_ref[pl.ds(h, h), :] + comm_ref[1].astype(jnp.float32)
        )

        send_ref[0] = out_ref[pl.ds(0, h), :].astype(jnp.bfloat16)
        send_ref[1] = out_ref[pl.ds(h, h), :].astype(jnp.bfloat16)
        rdma_a2 = pltpu.make_async_remote_copy(
            src_ref=send_ref.at[0],
            dst_ref=comm_ref.at[2],
            send_sem=send_sems.at[2],
            recv_sem=recv_sems.at[2],
            device_id=y_nbr,
            device_id_type=pl.DeviceIdType.MESH,
        )
        rdma_b2 = pltpu.make_async_remote_copy(
            src_ref=send_ref.at[1],
            dst_ref=comm_ref.at[3],
            send_sem=send_sems.at[3],
            recv_sem=recv_sems.at[3],
            device_id=x_nbr,
            device_id_type=pl.DeviceIdType.MESH,
        )
        rdma_a2.start()
        rdma_b2.start()
        rdma_a2.wait()
        rdma_b2.wait()
        out_ref[pl.ds(0, h), :] = (
            out_ref[pl.ds(0, h), :] + comm_ref[2].astype(jnp.float32)
        )
        out_ref[pl.ds(h, h), :] = (
            out_ref[pl.ds(h, h), :] + comm_ref[3].astype(jnp.float32)
        )

    return pl.pallas_call(
        body,
        out_shape=jax.ShapeDtypeStruct((m, n), jnp.float32),
        in_specs=[pl.BlockSpec(memory_space=pltpu.VMEM)],
        out_specs=pl.BlockSpec(memory_space=pltpu.VMEM),
        scratch_shapes=[
            pltpu.VMEM((2, h, n), jnp.bfloat16),
            pltpu.VMEM((4, h, n), jnp.bfloat16),
            pltpu.SemaphoreType.DMA((4,)),
            pltpu.SemaphoreType.DMA((4,)),
        ],
        compiler_params=pltpu.CompilerParams(collective_id=0),
    )(x)


# device time: 13856 ns/iter; 1.4350x vs baseline; 1.0312x over previous
import jax
import jax.numpy as jnp
from jax import lax
from jax.experimental import pallas as pl
from jax.experimental.pallas import tpu as pltpu


def kernel(x):
    m, n = x.shape
    h = m // 2

    def body(x_ref, out_ref, send_ref, comm_ref, send_sems, recv_sems):
        my_x = lax.axis_index("x")
        my_y = lax.axis_index("y")
        x_nbr = (1 - my_x, my_y)
        y_nbr = (my_x, 1 - my_y)

        def copy(src_slot, dst_slot, sem_slot, nbr):
            return pltpu.make_async_remote_copy(
                src_ref=send_ref.at[src_slot],
                dst_ref=comm_ref.at[dst_slot],
                send_sem=send_sems.at[sem_slot],
                recv_sem=recv_sems.at[sem_slot],
                device_id=nbr,
                device_id_type=pl.DeviceIdType.MESH,
            )

        barrier_sem = pltpu.get_barrier_semaphore()
        for nbr in (x_nbr, y_nbr):
            pl.semaphore_signal(
                barrier_sem, inc=1,
                device_id=nbr, device_id_type=pl.DeviceIdType.MESH,
            )
        send_ref[0] = x_ref[pl.ds(0, h), :].astype(jnp.bfloat16)
        send_ref[1] = x_ref[pl.ds(h, h), :].astype(jnp.bfloat16)
        pl.semaphore_wait(barrier_sem, 2)

        rdma_a1 = copy(0, 0, 0, x_nbr)
        rdma_b1 = copy(1, 1, 1, y_nbr)
        rdma_a1.start()
        rdma_b1.start()

        rdma_a1.wait_recv()
        send_ref[2] = send_ref[0] + comm_ref[0]
        rdma_a2 = copy(2, 2, 2, y_nbr)
        rdma_a2.start()

        rdma_b1.wait_recv()
        send_ref[3] = send_ref[1] + comm_ref[1]
        rdma_b2 = copy(3, 3, 3, x_nbr)
        rdma_b2.start()

        rdma_a2.wait_recv()
        out_ref[pl.ds(0, h), :] = (
            send_ref[2].astype(jnp.float32) + comm_ref[2].astype(jnp.float32)
        )
        rdma_b2.wait_recv()
        out_ref[pl.ds(h, h), :] = (
            send_ref[3].astype(jnp.float32) + comm_ref[3].astype(jnp.float32)
        )

        rdma_a1.wait_send()
        rdma_b1.wait_send()
        rdma_a2.wait_send()
        rdma_b2.wait_send()

    return pl.pallas_call(
        body,
        out_shape=jax.ShapeDtypeStruct((m, n), jnp.float32),
        in_specs=[pl.BlockSpec(memory_space=pltpu.VMEM)],
        out_specs=pl.BlockSpec(memory_space=pltpu.VMEM),
        scratch_shapes=[
            pltpu.VMEM((4, h, n), jnp.bfloat16),
            pltpu.VMEM((4, h, n), jnp.bfloat16),
            pltpu.SemaphoreType.DMA((4,)),
            pltpu.SemaphoreType.DMA((4,)),
        ],
        compiler_params=pltpu.CompilerParams(collective_id=0),
    )(x)


# device time: 12889 ns/iter; 1.5427x vs baseline; 1.0750x over previous
import jax
import jax.numpy as jnp
from jax import lax
from jax.experimental import pallas as pl
from jax.experimental.pallas import tpu as pltpu


def kernel(x):
    m, n = x.shape
    q = m // 4

    def body(x_ref, out_ref, send1_ref, recv1_ref, send2_ref, recv2_ref,
             send_sems1, recv_sems1, send_sems2, recv_sems2):
        my_x = lax.axis_index("x")
        my_y = lax.axis_index("y")
        x_nbr = (1 - my_x, my_y)
        y_nbr = (my_x, 1 - my_y)
        first_nbr = [x_nbr, x_nbr, y_nbr, y_nbr]
        second_nbr = [y_nbr, y_nbr, x_nbr, x_nbr]
        order = [0, 2, 1, 3]

        def copy(src, dst, ssem, rsem, nbr):
            return pltpu.make_async_remote_copy(
                src_ref=src, dst_ref=dst, send_sem=ssem, recv_sem=rsem,
                device_id=nbr, device_id_type=pl.DeviceIdType.MESH,
            )

        barrier_sem = pltpu.get_barrier_semaphore()
        for nbr in (x_nbr, y_nbr):
            pl.semaphore_signal(
                barrier_sem, inc=1,
                device_id=nbr, device_id_type=pl.DeviceIdType.MESH,
            )
        for i in range(4):
            send1_ref[i] = x_ref[pl.ds(i * q, q), :].astype(jnp.bfloat16)
        pl.semaphore_wait(barrier_sem, 2)

        p1 = [
            copy(send1_ref.at[i], recv1_ref.at[i],
                 send_sems1.at[i], recv_sems1.at[i], first_nbr[i])
            for i in range(4)
        ]
        for i in order:
            p1[i].start()

        p2 = [None] * 4
        for i in order:
            p1[i].wait_recv()
            send2_ref[i] = send1_ref[i] + recv1_ref[i]
            p2[i] = copy(send2_ref.at[i], recv2_ref.at[i],
                         send_sems2.at[i], recv_sems2.at[i], second_nbr[i])
            p2[i].start()

        for i in order:
            p2[i].wait_recv()
            out_ref[pl.ds(i * q, q), :] = (
                send2_ref[i].astype(jnp.float32)
                + recv2_ref[i].astype(jnp.float32)
            )

        for i in range(4):
            p1[i].wait_send()
            p2[i].wait_send()

    return pl.pallas_call(
        body,
        out_shape=jax.ShapeDtypeStruct((m, n), jnp.float32),
        in_specs=[pl.BlockSpec(memory_space=pltpu.VMEM)],
        out_specs=pl.BlockSpec(memory_space=pltpu.VMEM),
        scratch_shapes=[
            pltpu.VMEM((4, q, n), jnp.bfloat16),
            pltpu.VMEM((4, q, n), jnp.bfloat16),
            pltpu.VMEM((4, q, n), jnp.bfloat16),
            pltpu.VMEM((4, q, n), jnp.bfloat16),
            pltpu.SemaphoreType.DMA((4,)),
            pltpu.SemaphoreType.DMA((4,)),
            pltpu.SemaphoreType.DMA((4,)),
            pltpu.SemaphoreType.DMA((4,)),
        ],
        compiler_params=pltpu.CompilerParams(collective_id=0),
    )(x)
